# baseline (speedup 1.0000x reference)
import functools

import jax
import jax.numpy as jnp
from jax import lax
from jax.experimental import pallas as pl
from jax.experimental.pallas import tpu as pltpu
from jax.experimental.pallas import tpu_sc as plsc

N = 50000
D = 128
H = 128
HW = H // 2
S = 10

NW = 32
NPAD = 51200
C = 40
CS = C * S
CH0 = 80
CH1 = 0
MAXCH = max(CH0, CH1)
PAIR = (CH0 + CH1) * C
GSZ = (112, 96, 96, 96)
GOFF = (0, 112, 208, 304)

BM = 2000


def _pack_bf16_words(r_lo, r_hi):
    blo = lax.bitcast_convert_type(r_lo, jnp.int32) + jnp.int32(0x8000)
    bhi = lax.bitcast_convert_type(r_hi, jnp.int32) + jnp.int32(0x8000)
    lo = lax.shift_right_logical(blo, 16)
    hi = bhi & jnp.int32(-65536)
    return hi | lo


def _mm_body(f_ref, wsl_ref, wsh_ref, wnl_ref, wnh_ref, ps_ref, pn_ref):
    f = f_ref[...]
    ps_ref[...] = _pack_bf16_words(
        jnp.dot(f, wsl_ref[...], preferred_element_type=jnp.float32),
        jnp.dot(f, wsh_ref[...], preferred_element_type=jnp.float32))
    pn_ref[...] = _pack_bf16_words(
        jnp.dot(f, wnl_ref[...], preferred_element_type=jnp.float32),
        jnp.dot(f, wnh_ref[...], preferred_element_type=jnp.float32))


def _project(feature, w_self_lo, w_self_hi, w_neigh_lo, w_neigh_hi):
    wspec = pl.BlockSpec((D, HW), lambda i: (0, 0))
    return pl.pallas_call(
        _mm_body,
        grid=(N // BM,),
        in_specs=[pl.BlockSpec((BM, D), lambda i: (i, 0)),
                  wspec, wspec, wspec, wspec],
        out_specs=[
            pl.BlockSpec((BM, HW), lambda i: (i, 0)),
            pl.BlockSpec((BM, HW), lambda i: (i, 0)),
        ],
        out_shape=[
            jax.ShapeDtypeStruct((N, HW), jnp.int32),
            jax.ShapeDtypeStruct((N, HW), jnp.int32),
        ],
    )(feature, w_self_lo, w_self_hi, w_neigh_lo, w_neigh_hi)


_mesh = plsc.VectorSubcoreMesh(core_axis_name="c", subcore_axis_name="s")


@functools.partial(
    pl.kernel,
    mesh=_mesh,
    compiler_params=pltpu.CompilerParams(use_tc_tiling_on_sc=False),
    out_type=jax.ShapeDtypeStruct((NPAD, H), jnp.float32),
    scratch_types=[
        pltpu.VMEM((MAXCH * C,), jnp.int32),
        pltpu.VMEM((MAXCH * CS,), jnp.int32),
        pltpu.VMEM((2, C, HW), jnp.int32),
        pltpu.VMEM((2, CS, HW), jnp.int32),
        pltpu.VMEM((2, C, H), jnp.float32),
        pltpu.SemaphoreType.DMA,
        pltpu.SemaphoreType.DMA,
        pltpu.SemaphoreType.DMA,
        pltpu.SemaphoreType.DMA,
    ],
)
def _sc_agg(nodes_hbm, nidx_hbm, ps_hbm, pn_hbm, out_hbm,
            sidx_v, nidx_v, srows_v, nrows_v, outb_v,
            sem_g0, sem_g1, sem_o0, sem_o1):
    cid = lax.axis_index("c")
    sid = lax.axis_index("s")
    base = sid * PAIR + cid * (CH0 * C)
    my_ch = jnp.where(cid == 0, CH0, CH1)
    sem_g = (sem_g0, sem_g1)
    sem_o = (sem_o0, sem_o1)
    himask = jnp.int32(-65536)

    @pl.when(my_ch > 0)
    def _():
        pltpu.sync_copy(nodes_hbm.at[pl.ds(base, MAXCH * C)], sidx_v)
        pltpu.sync_copy(nidx_hbm.at[pl.ds(base * S, MAXCH * CS)], nidx_v)

    def issue(k, b):
        pltpu.async_copy(
            ps_hbm.at[sidx_v.at[pl.ds(k * C, C)]], srows_v.at[b], sem_g[b])
        for g in range(4):
            pltpu.async_copy(
                pn_hbm.at[nidx_v.at[pl.ds(k * CS + GOFF[g], GSZ[g])]],
                nrows_v.at[b, pl.ds(GOFF[g], GSZ[g])], sem_g[b])

    def wait_gathers(b):
        pltpu.make_async_copy(
            ps_hbm.at[pl.ds(0, C)], srows_v.at[b], sem_g[b]).wait()
        for g in range(4):
            pltpu.make_async_copy(
                pn_hbm.at[pl.ds(0, GSZ[g])],
                nrows_v.at[b, pl.ds(GOFF[g], GSZ[g])], sem_g[b]).wait()

    @pl.when(my_ch > 0)
    def _():
        issue(0, 0)

    def pair_body(it, carry):
        for b in range(2):
            k = it * 2 + b
            wait_gathers(b)

            @pl.when(k + 1 < my_ch)
            def _():
                issue(k + 1, 1 - b)

            @pl.when(k >= 2)
            def _():
                pltpu.make_async_copy(
                    out_hbm.at[pl.ds(0, C)], outb_v.at[b], sem_o[b]).wait()

            def node_body(i, c):
                r0 = i * S
                for g in range(4):
                    gsl = pl.ds(g * 16, 16)
                    w = srows_v[b, i, gsl]
                    acc_e = lax.bitcast_convert_type(w << 16, jnp.float32)
                    acc_o = lax.bitcast_convert_type(w & himask, jnp.float32)
                    for s in range(S):
                        w = nrows_v[b, r0 + s, gsl]
                        acc_e = acc_e + lax.bitcast_convert_type(w << 16, jnp.float32)
                        acc_o = acc_o + lax.bitcast_convert_type(w & himask, jnp.float32)
                    outb_v[b, i, pl.ds(g * 32, 16)] = jnp.maximum(acc_e, 0.0)
                    outb_v[b, i, pl.ds(g * 32 + 16, 16)] = jnp.maximum(acc_o, 0.0)
                return c

            lax.fori_loop(0, C, node_body, 0, unroll=False)
            pltpu.async_copy(
                outb_v.at[b], out_hbm.at[pl.ds(base + k * C, C)], sem_o[b])
        return carry

    lax.fori_loop(0, my_ch // 2, pair_body, 0, unroll=False)
    for b in range(2):
        @pl.when(my_ch > 0)
        def _():
            pltpu.make_async_copy(
                out_hbm.at[pl.ds(0, C)], outb_v.at[b], sem_o[b]).wait()


_LO = [32 * g + l for g in range(4) for l in range(16)]
_HI = [32 * g + 16 + l for g in range(4) for l in range(16)]


def kernel(nodes, neigh_idx, feature, weight):
    lo = jnp.array(_LO, dtype=jnp.int32)
    hi = jnp.array(_HI, dtype=jnp.int32)
    w_self = weight[:D]
    w_neigh = weight[D:] * (1.0 / S)
    ps, pn = _project(feature, w_self[:, lo], w_self[:, hi],
                      w_neigh[:, lo], w_neigh[:, hi])
    nodes_p = jnp.concatenate(
        [nodes.astype(jnp.int32), jnp.zeros((NPAD - N,), jnp.int32)])
    nidx_p = jnp.concatenate(
        [neigh_idx.reshape(-1).astype(jnp.int32),
         jnp.zeros(((NPAD - N) * S,), jnp.int32)])
    out = _sc_agg(nodes_p, nidx_p, ps, pn)
    return out[:N]


# speedup vs baseline: 1.2469x; 1.2469x over previous
import functools

import jax
import jax.numpy as jnp
from jax import lax
from jax.experimental import pallas as pl
from jax.experimental.pallas import tpu as pltpu
from jax.experimental.pallas import tpu_sc as plsc

N = 50000
D = 128
H = 128
HW = H // 2
S = 10

NW = 32
NPAD = 51200
C = 40
CS = C * S
CH0 = 58
CH1 = 22
MAXCH = max(CH0, CH1)
PAIR = (CH0 + CH1) * C
GSZ = (112, 96, 96, 96)
GOFF = (0, 112, 208, 304)

BM = 2000


def _pack_bf16_words(r_lo, r_hi):
    blo = lax.bitcast_convert_type(r_lo, jnp.int32) + jnp.int32(0x8000)
    bhi = lax.bitcast_convert_type(r_hi, jnp.int32) + jnp.int32(0x8000)
    lo = lax.shift_right_logical(blo, 16)
    hi = bhi & jnp.int32(-65536)
    return hi | lo


def _mm_body(f_ref, wsl_ref, wsh_ref, wnl_ref, wnh_ref, ps_ref, pn_ref):
    f = f_ref[...]
    ps_ref[...] = _pack_bf16_words(
        jnp.dot(f, wsl_ref[...], preferred_element_type=jnp.float32),
        jnp.dot(f, wsh_ref[...], preferred_element_type=jnp.float32))
    pn_ref[...] = _pack_bf16_words(
        jnp.dot(f, wnl_ref[...], preferred_element_type=jnp.float32),
        jnp.dot(f, wnh_ref[...], preferred_element_type=jnp.float32))


def _project(feature, w_self_lo, w_self_hi, w_neigh_lo, w_neigh_hi):
    wspec = pl.BlockSpec((D, HW), lambda i: (0, 0))
    return pl.pallas_call(
        _mm_body,
        grid=(N // BM,),
        in_specs=[pl.BlockSpec((BM, D), lambda i: (i, 0)),
                  wspec, wspec, wspec, wspec],
        out_specs=[
            pl.BlockSpec((BM, HW), lambda i: (i, 0)),
            pl.BlockSpec((BM, HW), lambda i: (i, 0)),
        ],
        out_shape=[
            jax.ShapeDtypeStruct((N, HW), jnp.int32),
            jax.ShapeDtypeStruct((N, HW), jnp.int32),
        ],
    )(feature, w_self_lo, w_self_hi, w_neigh_lo, w_neigh_hi)


_mesh = plsc.VectorSubcoreMesh(core_axis_name="c", subcore_axis_name="s")


@functools.partial(
    pl.kernel,
    mesh=_mesh,
    compiler_params=pltpu.CompilerParams(use_tc_tiling_on_sc=False),
    out_type=jax.ShapeDtypeStruct((NPAD, H), jnp.float32),
    scratch_types=[
        pltpu.VMEM((MAXCH * C,), jnp.int32),
        pltpu.VMEM((MAXCH * CS,), jnp.int32),
        pltpu.VMEM((2, C, HW), jnp.int32),
        pltpu.VMEM((2, CS, HW), jnp.int32),
        pltpu.VMEM((2, C, H), jnp.float32),
        pltpu.SemaphoreType.DMA,
        pltpu.SemaphoreType.DMA,
        pltpu.SemaphoreType.DMA,
        pltpu.SemaphoreType.DMA,
    ],
)
def _sc_agg(nodes_hbm, nidx_hbm, ps_hbm, pn_hbm, out_hbm,
            sidx_v, nidx_v, srows_v, nrows_v, outb_v,
            sem_g0, sem_g1, sem_o0, sem_o1):
    cid = lax.axis_index("c")
    sid = lax.axis_index("s")
    base = sid * PAIR + cid * (CH0 * C)
    my_ch = jnp.where(cid == 0, CH0, CH1)
    sem_g = (sem_g0, sem_g1)
    sem_o = (sem_o0, sem_o1)
    himask = jnp.int32(-65536)

    @pl.when(my_ch > 0)
    def _():
        pltpu.sync_copy(nodes_hbm.at[pl.ds(base, MAXCH * C)], sidx_v)
        pltpu.sync_copy(nidx_hbm.at[pl.ds(base * S, MAXCH * CS)], nidx_v)

    def issue(k, b):
        pltpu.async_copy(
            ps_hbm.at[sidx_v.at[pl.ds(k * C, C)]], srows_v.at[b], sem_g[b])
        pltpu.async_copy(
            pn_hbm.at[nidx_v.at[pl.ds(k * CS, CS)]], nrows_v.at[b], sem_g[b])

    def wait_gathers(b):
        pltpu.make_async_copy(
            ps_hbm.at[pl.ds(0, C)], srows_v.at[b], sem_g[b]).wait()
        pltpu.make_async_copy(
            pn_hbm.at[pl.ds(0, CS)], nrows_v.at[b], sem_g[b]).wait()

    @pl.when(my_ch > 0)
    def _():
        issue(0, 0)

    def pair_body(it, carry):
        for b in range(2):
            k = it * 2 + b
            wait_gathers(b)

            @pl.when(k + 1 < my_ch)
            def _():
                issue(k + 1, 1 - b)

            @pl.when(k >= 2)
            def _():
                pltpu.make_async_copy(
                    out_hbm.at[pl.ds(0, C)], outb_v.at[b], sem_o[b]).wait()

            def node_body(i, c):
                r0 = i * S
                for g in range(4):
                    gsl = pl.ds(g * 16, 16)
                    w = srows_v[b, i, gsl]
                    acc_e = lax.bitcast_convert_type(w << 16, jnp.float32)
                    acc_o = lax.bitcast_convert_type(w & himask, jnp.float32)
                    for s in range(S):
                        w = nrows_v[b, r0 + s, gsl]
                        acc_e = acc_e + lax.bitcast_convert_type(w << 16, jnp.float32)
                        acc_o = acc_o + lax.bitcast_convert_type(w & himask, jnp.float32)
                    outb_v[b, i, pl.ds(g * 32, 16)] = jnp.maximum(acc_e, 0.0)
                    outb_v[b, i, pl.ds(g * 32 + 16, 16)] = jnp.maximum(acc_o, 0.0)
                return c

            lax.fori_loop(0, C, node_body, 0, unroll=False)
            pltpu.async_copy(
                outb_v.at[b], out_hbm.at[pl.ds(base + k * C, C)], sem_o[b])
        return carry

    lax.fori_loop(0, my_ch // 2, pair_body, 0, unroll=False)
    for b in range(2):
        @pl.when(my_ch > 0)
        def _():
            pltpu.make_async_copy(
                out_hbm.at[pl.ds(0, C)], outb_v.at[b], sem_o[b]).wait()


_LO = [32 * g + l for g in range(4) for l in range(16)]
_HI = [32 * g + 16 + l for g in range(4) for l in range(16)]


def kernel(nodes, neigh_idx, feature, weight):
    lo = jnp.array(_LO, dtype=jnp.int32)
    hi = jnp.array(_HI, dtype=jnp.int32)
    w_self = weight[:D]
    w_neigh = weight[D:] * (1.0 / S)
    ps, pn = _project(feature, w_self[:, lo], w_self[:, hi],
                      w_neigh[:, lo], w_neigh[:, hi])
    nodes_p = jnp.concatenate(
        [nodes.astype(jnp.int32), jnp.zeros((NPAD - N,), jnp.int32)])
    nidx_p = jnp.concatenate(
        [neigh_idx.reshape(-1).astype(jnp.int32),
         jnp.zeros(((NPAD - N) * S,), jnp.int32)])
    out = _sc_agg(nodes_p, nidx_p, ps, pn)
    return out[:N]
